# Initial kernel scaffold; baseline (speedup 1.0000x reference)
import jax
import jax.numpy as jnp
from jax.experimental import pallas as pl


def kernel(inputs, token_table, pos_table):
    raise NotImplementedError("write your pallas kernel here")


# speedup vs baseline: 2.3699x; 2.3699x over previous
import functools

import jax
import jax.numpy as jnp
from jax import lax
from jax.experimental import pallas as pl
from jax.experimental.pallas import tpu as pltpu, tpu_sc as plsc

SEQ_LEN = 200
VOCAB = 100000
EMBED_DIM = 64
BATCH = 4096

NUM_CORES = 2
NUM_SUBCORES = 16
NUM_WORKERS = NUM_CORES * NUM_SUBCORES
ROWS_PER_WORKER = BATCH // NUM_WORKERS
IDX_PER_WORKER = ROWS_PER_WORKER * SEQ_LEN
LANES = 16
CHUNKS_PER_ROW = EMBED_DIM // LANES
SCALE = 8.0

GATHER_SPLITS = ((0, 128), (128, 72))


def _sc_kernel(idx_hbm, table_hbm, pos_hbm, out_hbm, idx_v, pos_v, rows_v, sem):
    wid = lax.axis_index("s") * NUM_CORES + lax.axis_index("c")
    row_base = wid * ROWS_PER_WORKER
    idx_base = wid * IDX_PER_WORKER

    pltpu.sync_copy(idx_hbm.at[pl.ds(idx_base, IDX_PER_WORKER)], idx_v)
    pltpu.sync_copy(pos_hbm, pos_v)

    def per_row(t, _):
        copies = []
        for off, n in GATHER_SPLITS:
            copies.append(
                pltpu.async_copy(
                    table_hbm.at[idx_v.at[pl.ds(t * SEQ_LEN + off, n)]],
                    rows_v.at[pl.ds(off, n)],
                    sem,
                )
            )
        for c in copies:
            c.wait()

        def per_seq(i, _):
            for j in range(CHUNKS_PER_ROW):
                sl = pl.ds(j * LANES, LANES)
                rows_v[i, sl] = rows_v[i, sl] * SCALE + pos_v[i, sl]
            return ()

        lax.fori_loop(0, SEQ_LEN, per_seq, (), unroll=2)

        pltpu.sync_copy(rows_v, out_hbm.at[row_base + t])
        return ()

    lax.fori_loop(0, ROWS_PER_WORKER, per_row, ())


@jax.jit
def kernel(inputs, token_table, pos_table):
    mesh = plsc.VectorSubcoreMesh(core_axis_name="c", subcore_axis_name="s")
    f = pl.kernel(
        _sc_kernel,
        out_type=jax.ShapeDtypeStruct((BATCH, SEQ_LEN, EMBED_DIM), jnp.float32),
        mesh=mesh,
        scratch_types=[
            pltpu.VMEM((IDX_PER_WORKER,), jnp.int32),
            pltpu.VMEM((SEQ_LEN, EMBED_DIM), jnp.float32),
            pltpu.VMEM((SEQ_LEN, EMBED_DIM), jnp.float32),
            pltpu.SemaphoreType.DMA,
        ],
        compiler_params=pltpu.CompilerParams(use_tc_tiling_on_sc=False),
    )
    return f(inputs.reshape(-1), token_table, pos_table)


# speedup vs baseline: 3.4739x; 1.4659x over previous
import jax
import jax.numpy as jnp
from jax import lax
from jax.experimental import pallas as pl
from jax.experimental.pallas import tpu as pltpu, tpu_sc as plsc

SEQ_LEN = 200
VOCAB = 100000
EMBED_DIM = 64
BATCH = 4096

NUM_CORES = 2
NUM_SUBCORES = 16
NUM_WORKERS = NUM_CORES * NUM_SUBCORES
ROWS_PER_WORKER = BATCH // NUM_WORKERS
IDX_PER_WORKER = ROWS_PER_WORKER * SEQ_LEN
LANES = 16
CHUNKS_PER_ROW = EMBED_DIM // LANES
SCALE = 8.0
NBUF = 4

GATHER_SPLITS = ((0, 128), (128, 72))


def _sc_kernel(idx_hbm, table_hbm, pos_hbm, out_hbm, idx_v, pos_v, rows_v, gsems, wsems):
    wid = lax.axis_index("s") * NUM_CORES + lax.axis_index("c")
    row_base = wid * ROWS_PER_WORKER
    idx_base = wid * IDX_PER_WORKER

    pltpu.sync_copy(idx_hbm.at[pl.ds(idx_base, IDX_PER_WORKER)], idx_v)
    pltpu.sync_copy(pos_hbm, pos_v)

    def issue_gather(t, k):
        for off, n in GATHER_SPLITS:
            pltpu.async_copy(
                table_hbm.at[idx_v.at[pl.ds(t * SEQ_LEN + off, n)]],
                rows_v.at[k, pl.ds(off, n)],
                gsems[k],
            )

    def wait_gather(t, k):
        for off, n in GATHER_SPLITS:
            pltpu.make_async_copy(
                table_hbm.at[idx_v.at[pl.ds(t * SEQ_LEN + off, n)]],
                rows_v.at[k, pl.ds(off, n)],
                gsems[k],
            ).wait()

    def issue_write(t, k):
        pltpu.async_copy(rows_v.at[k], out_hbm.at[row_base + t], wsems[k])

    def wait_write(t, k):
        pltpu.make_async_copy(
            rows_v.at[k], out_hbm.at[row_base + t], wsems[k]
        ).wait()

    for k in range(NBUF - 1):
        issue_gather(k, k)

    def ring_body(g, _):
        u_outer = g * NBUF
        for k in range(NBUF):
            u = u_outer + k
            wait_gather(u, k)

            def per_seq(i, _):
                for j in range(CHUNKS_PER_ROW):
                    sl = pl.ds(j * LANES, LANES)
                    rows_v[k, i, sl] = rows_v[k, i, sl] * SCALE + pos_v[i, sl]
                return ()

            lax.fori_loop(0, SEQ_LEN, per_seq, (), unroll=2)

            issue_write(u, k)

            r = u + NBUF - 1
            kr = (k - 1) % NBUF

            @pl.when(r < ROWS_PER_WORKER)
            def _():
                @pl.when(u >= 1)
                def _():
                    wait_write(u - 1, kr)

                issue_gather(r, kr)

        return ()

    lax.fori_loop(0, ROWS_PER_WORKER // NBUF, ring_body, ())

    for k in range(NBUF):
        wait_write(ROWS_PER_WORKER - NBUF + k, k)


@jax.jit
def kernel(inputs, token_table, pos_table):
    mesh = plsc.VectorSubcoreMesh(core_axis_name="c", subcore_axis_name="s")
    f = pl.kernel(
        _sc_kernel,
        out_type=jax.ShapeDtypeStruct((BATCH, SEQ_LEN, EMBED_DIM), jnp.float32),
        mesh=mesh,
        scratch_types=[
            pltpu.VMEM((IDX_PER_WORKER,), jnp.int32),
            pltpu.VMEM((SEQ_LEN, EMBED_DIM), jnp.float32),
            pltpu.VMEM((NBUF, SEQ_LEN, EMBED_DIM), jnp.float32),
            [pltpu.SemaphoreType.DMA] * NBUF,
            [pltpu.SemaphoreType.DMA] * NBUF,
        ],
        compiler_params=pltpu.CompilerParams(use_tc_tiling_on_sc=False),
    )
    return f(inputs.reshape(-1), token_table, pos_table)
